# baseline (speedup 1.0000x reference)
import functools

import jax
import jax.numpy as jnp
from jax import lax
from jax.experimental import pallas as pl
from jax.experimental import shard_map
from jax.sharding import Mesh, PartitionSpec as P

_E = 1024
_D = 64
_B = 16
_N = 1024
_TOTAL = _B * _N


def _vq_kernel(nsteps, x_ref, emb_ref, acc_ref, qst_ref, counts_ref, enc_ref):
    b = pl.program_id(0)

    x = x_ref[0]
    emb = emb_ref[...]

    s = jax.lax.dot_general(emb, x, (((1,), (0,)), ((), ())),
                            preferred_element_type=jnp.float32)
    x2 = jnp.sum(x * x, axis=0, keepdims=True)
    e2 = jnp.sum(emb * emb, axis=1, keepdims=True)
    dist = (x2 + e2) - 2.0 * s

    m = jnp.min(dist, axis=0, keepdims=True)
    e_iota = lax.broadcasted_iota(jnp.int32, (_E, _N), 0)
    idx_row = jnp.min(jnp.where(dist == m, e_iota, _E), axis=0,
                      keepdims=True)

    enc_t = (e_iota == idx_row).astype(jnp.float32)

    idx_col = jnp.transpose(idx_row, (1, 0))

    e_lane = lax.broadcasted_iota(jnp.int32, (_N, _E), 1)
    enc_ref[...] = (idx_col == e_lane).astype(jnp.float32)

    q = jax.lax.dot_general(emb, enc_t, (((0,), (0,)), ((), ())),
                            preferred_element_type=jnp.float32)
    d_qx = q - x
    qst_ref[0] = x + d_qx

    @pl.when(b == 0)
    def _init():
        acc_ref[...] = jnp.zeros_like(acc_ref)
        counts_ref[...] = jnp.zeros_like(counts_ref)

    acc_ref[...] += jnp.reshape(jnp.sum(d_qx * d_qx), (1, 1))
    counts_ref[...] += jnp.sum(enc_t, axis=1, keepdims=True)

    del nsteps


def _vq_shard(x3s, emb):
    nsteps = x3s.shape[0]
    acc, qst, counts, enc = pl.pallas_call(
        functools.partial(_vq_kernel, nsteps),
        grid=(nsteps,),
        in_specs=[
            pl.BlockSpec((1, _D, _N), lambda b: (b, 0, 0)),
            pl.BlockSpec((_E, _D), lambda b: (0, 0)),
        ],
        out_specs=[
            pl.BlockSpec((1, 1), lambda b: (0, 0)),
            pl.BlockSpec((1, _D, _N), lambda b: (b, 0, 0)),
            pl.BlockSpec((_E, 1), lambda b: (0, 0)),
            pl.BlockSpec((_N, _E), lambda b: (b, 0)),
        ],
        out_shape=[
            jax.ShapeDtypeStruct((1, 1), jnp.float32),
            jax.ShapeDtypeStruct((nsteps, _D, _N), jnp.float32),
            jax.ShapeDtypeStruct((_E, 1), jnp.float32),
            jax.ShapeDtypeStruct((nsteps * _N, _E), jnp.float32),
        ],
    )(x3s, emb)

    acc_g = lax.psum(acc, "x")
    counts_g = lax.psum(counts, "x")
    loss = 0.25 * (acc_g[0, 0] / float(_TOTAL * _D))
    p = counts_g[:, 0] * (1.0 / float(_TOTAL))
    perplexity = jnp.exp(-jnp.sum(p * jnp.log(p + 1e-10)))
    return loss, qst, perplexity, enc


def _build_sharded(ndev):
    devs = jax.devices()[:ndev]
    mesh = Mesh(devs, ("x",))
    return shard_map.shard_map(
        _vq_shard,
        mesh=mesh,
        in_specs=(P("x", None, None), P(None, None)),
        out_specs=(P(), P("x", None, None), P(), P("x", None)),
        check_rep=False,
    )


def kernel(inputs, embedding_weight):
    ndev = 2 if len(jax.devices()) >= 2 else 1
    x3 = inputs.reshape(_B, _D, _N)
    loss, qst3, perplexity, enc = _build_sharded(ndev)(x3, embedding_weight)
    return (loss, qst3.reshape(_B, _D, 32, 32), perplexity, enc)


# speedup vs baseline: 8.8914x; 8.8914x over previous
import functools

import jax
import jax.numpy as jnp
from jax import lax
from jax.experimental import pallas as pl
from jax.experimental.pallas import tpu as pltpu

_E = 1024
_D = 64
_B = 16
_N = 1024
_SLICES_PER_STEP = 2
_STEPS = _B // _SLICES_PER_STEP
_TOTAL = _B * _N


def _vq_kernel(x_ref, emb_ref, loss_ref, qst_ref, perp_ref, enc_ref,
               acc_ref, counts_ref, e2_ref):
    step = pl.program_id(0)
    emb = emb_ref[...]

    @pl.when(step == 0)
    def _init():
        acc_ref[0, 0] = 0.0
        counts_ref[...] = jnp.zeros_like(counts_ref)
        e2_ref[...] = jnp.sum(emb * emb, axis=1, keepdims=True)

    e2 = e2_ref[...]
    ones_col = jnp.ones((_N, 1), jnp.float32)

    for i in range(_SLICES_PER_STEP):
        x = x_ref[i]

        s = jax.lax.dot_general(emb, x, (((1,), (0,)), ((), ())),
                                preferred_element_type=jnp.float32)
        x2 = jnp.sum(x * x, axis=0, keepdims=True)
        dist = (x2 + e2) - 2.0 * s

        m = jnp.min(dist, axis=0, keepdims=True)
        e_iota = lax.broadcasted_iota(jnp.int32, (_E, _N), 0)
        idx_row = jnp.min(jnp.where(dist == m, e_iota, _E), axis=0,
                          keepdims=True)

        enc_t = (e_iota == idx_row).astype(jnp.float32)

        idx_col = jnp.transpose(idx_row, (1, 0))

        e_lane = lax.broadcasted_iota(jnp.int32, (_N, _E), 1)
        enc_ref[pl.ds(i * _N, _N), :] = (idx_col == e_lane).astype(jnp.float32)

        q = jax.lax.dot_general(emb, enc_t, (((0,), (0,)), ((), ())),
                                preferred_element_type=jnp.float32)
        d_qx = q - x
        qst_ref[i] = x + d_qx

        acc_ref[0, 0] += jnp.sum(d_qx * d_qx)
        counts_ref[...] += jax.lax.dot_general(
            enc_t, ones_col, (((1,), (0,)), ((), ())),
            preferred_element_type=jnp.float32)

    @pl.when(step == _STEPS - 1)
    def _fini():
        loss_ref[...] = jnp.reshape(
            0.25 * (acc_ref[0, 0] / float(_TOTAL * _D)), (1, 1))
        p = counts_ref[...] * (1.0 / float(_TOTAL))
        ent = p * jnp.log(p + 1e-10)
        perp_ref[...] = jnp.reshape(jnp.exp(-jnp.sum(ent)), (1, 1))


@functools.partial(jax.jit, static_argnames=())
def kernel(inputs, embedding_weight):
    x3 = inputs.reshape(_B, _D, _N)

    loss2d, qst3, perp2d, enc = pl.pallas_call(
        _vq_kernel,
        grid=(_STEPS,),
        in_specs=[
            pl.BlockSpec((_SLICES_PER_STEP, _D, _N), lambda b: (b, 0, 0)),
            pl.BlockSpec((_E, _D), lambda b: (0, 0)),
        ],
        out_specs=[
            pl.BlockSpec((1, 1), lambda b: (0, 0)),
            pl.BlockSpec((_SLICES_PER_STEP, _D, _N), lambda b: (b, 0, 0)),
            pl.BlockSpec((1, 1), lambda b: (0, 0)),
            pl.BlockSpec((_SLICES_PER_STEP * _N, _E), lambda b: (b, 0)),
        ],
        out_shape=[
            jax.ShapeDtypeStruct((1, 1), jnp.float32),
            jax.ShapeDtypeStruct((_B, _D, _N), jnp.float32),
            jax.ShapeDtypeStruct((1, 1), jnp.float32),
            jax.ShapeDtypeStruct((_TOTAL, _E), jnp.float32),
        ],
        scratch_shapes=[
            pltpu.SMEM((1, 1), jnp.float32),
            pltpu.VMEM((_E, 1), jnp.float32),
            pltpu.VMEM((_E, 1), jnp.float32),
        ],
    )(x3, embedding_weight)

    return (loss2d[0, 0],
            qst3.reshape(_B, _D, 32, 32),
            perp2d[0, 0],
            enc)


# speedup vs baseline: 10.7234x; 1.2060x over previous
import functools

import jax
import jax.numpy as jnp
from jax import lax
from jax.experimental import pallas as pl
from jax.experimental.pallas import tpu as pltpu

_E = 1024
_D = 64
_B = 16
_N = 1024
_SLICES_PER_STEP = 1
_STEPS = _B // _SLICES_PER_STEP
_TOTAL = _B * _N


def _vq_kernel(x_ref, emb_ref, loss_ref, qst_ref, perp_ref, enc_ref,
               acc_ref, counts_ref, e2_ref):
    step = pl.program_id(0)
    emb = emb_ref[...]

    @pl.when(step == 0)
    def _init():
        acc_ref[0, 0] = 0.0
        counts_ref[...] = jnp.zeros_like(counts_ref)
        e2_ref[...] = jnp.sum(emb * emb, axis=1, keepdims=True)

    e2 = e2_ref[...]

    for i in range(_SLICES_PER_STEP):
        x = x_ref[i]

        s = jax.lax.dot_general(emb, x, (((1,), (0,)), ((), ())),
                                preferred_element_type=jnp.float32)
        x2 = jnp.sum(x * x, axis=0, keepdims=True)
        dist = (x2 + e2) - 2.0 * s

        e_iota = lax.broadcasted_iota(jnp.int32, (_E, _N), 0)
        idx_row = jnp.argmin(dist, axis=0, keepdims=True)

        enc_t = (e_iota == idx_row).astype(jnp.float32)

        idx_col = jnp.transpose(idx_row, (1, 0))

        e_lane = lax.broadcasted_iota(jnp.int32, (_N, _E), 1)
        enc_ref[pl.ds(i * _N, _N), :] = (idx_col == e_lane).astype(jnp.float32)

        q = jax.lax.dot_general(emb, enc_t, (((0,), (0,)), ((), ())),
                                preferred_element_type=jnp.float32)
        d_qx = q - x
        qst_ref[i] = x + d_qx

        acc_ref[0, 0] += jnp.sum(d_qx * d_qx)
        counts_ref[...] += jnp.sum(enc_t, axis=1, keepdims=True)

    @pl.when(step == _STEPS - 1)
    def _fini():
        loss_ref[...] = jnp.reshape(
            0.25 * (acc_ref[0, 0] / float(_TOTAL * _D)), (1, 1))
        p = counts_ref[...] * (1.0 / float(_TOTAL))
        ent = p * jnp.log(p + 1e-10)
        perp_ref[...] = jnp.reshape(jnp.exp(-jnp.sum(ent)), (1, 1))


@functools.partial(jax.jit, static_argnames=())
def kernel(inputs, embedding_weight):
    x3 = inputs.reshape(_B, _D, _N)

    loss2d, qst3, perp2d, enc = pl.pallas_call(
        _vq_kernel,
        grid=(_STEPS,),
        in_specs=[
            pl.BlockSpec((_SLICES_PER_STEP, _D, _N), lambda b: (b, 0, 0)),
            pl.BlockSpec((_E, _D), lambda b: (0, 0)),
        ],
        out_specs=[
            pl.BlockSpec((1, 1), lambda b: (0, 0)),
            pl.BlockSpec((_SLICES_PER_STEP, _D, _N), lambda b: (b, 0, 0)),
            pl.BlockSpec((1, 1), lambda b: (0, 0)),
            pl.BlockSpec((_SLICES_PER_STEP * _N, _E), lambda b: (b, 0)),
        ],
        out_shape=[
            jax.ShapeDtypeStruct((1, 1), jnp.float32),
            jax.ShapeDtypeStruct((_B, _D, _N), jnp.float32),
            jax.ShapeDtypeStruct((1, 1), jnp.float32),
            jax.ShapeDtypeStruct((_TOTAL, _E), jnp.float32),
        ],
        scratch_shapes=[
            pltpu.SMEM((1, 1), jnp.float32),
            pltpu.VMEM((_E, 1), jnp.float32),
            pltpu.VMEM((_E, 1), jnp.float32),
        ],
    )(x3, embedding_weight)

    return (loss2d[0, 0],
            qst3.reshape(_B, _D, 32, 32),
            perp2d[0, 0],
            enc)
